# baseline (speedup 1.0000x reference)
import functools

import jax
import jax.numpy as jnp
from jax import lax
from jax.experimental import pallas as pl
from jax.experimental.pallas import tpu as pltpu
from jax.experimental.pallas import tpu_sc as plsc

N = 10000
E = 320000
D = 128

NC = 2
NS = 16
NW = NC * NS

CHUNK = 128
K = 79
KD = 80
KS = 81
EPT = K * CHUNK
E_PAD = NW * EPT
NP = 10112
ROWS_PER_TILE = NP // NS
CW = 128

_mesh = plsc.VectorSubcoreMesh(
    core_axis_name="c", subcore_axis_name="s", num_cores=NC, num_subcores=NS)


PBITS = 14
PMASK = (1 << PBITS) - 1


def _unpack_src(pk_v, j, buf):
    for k in range(CHUNK // 16):
        pv = pk_v[j, pl.ds(k * 16, 16)]
        buf[pl.ds(k * 16, 16)] = lax.shift_right_logical(pv, PBITS)


def _unpack_dst(pk_v, j, buf):
    for k in range(CHUNK // 16):
        pv = pk_v[j, pl.ds(k * 16, 16)]
        buf[pl.ds(k * 16, 16)] = lax.bitwise_and(pv, PMASK)


def _sc_agg_body(x_hbm, pk_hbm, zagg_hbm, agg_out, pk_v, src_a, src_b,
                 dst_a, dst_b, rows_v0, rows_v1, agg_sh, sem0, sem1):
    cid = lax.axis_index("c")
    sid = lax.axis_index("s")
    wid = sid * NC + cid

    zbase = sid * ROWS_PER_TILE
    pltpu.sync_copy(zagg_hbm.at[pl.ds(zbase, ROWS_PER_TILE)],
                    agg_sh.at[pl.ds(zbase, ROWS_PER_TILE)])

    pltpu.sync_copy(pk_hbm.at[wid], pk_v)
    plsc.subcore_barrier()

    _unpack_src(pk_v, 0, src_a)
    pltpu.async_copy(x_hbm.at[src_a], rows_v0, sem0)

    def body(i, carry):
        j0 = 2 * i
        pltpu.make_async_copy(x_hbm.at[src_a], rows_v0, sem0).wait()
        _unpack_src(pk_v, j0 + 1, src_b)
        pltpu.async_copy(x_hbm.at[src_b], rows_v1, sem1)
        _unpack_dst(pk_v, j0, dst_a)
        pltpu.sync_copy(rows_v0, agg_sh.at[dst_a], add=True)
        pltpu.make_async_copy(x_hbm.at[src_b], rows_v1, sem1).wait()
        _unpack_src(pk_v, j0 + 2, src_a)
        pltpu.async_copy(x_hbm.at[src_a], rows_v0, sem0)
        _unpack_dst(pk_v, j0 + 1, dst_b)
        pltpu.sync_copy(rows_v1, agg_sh.at[dst_b], add=True)
        return carry

    lax.fori_loop(0, KD // 2, body, 0)
    pltpu.make_async_copy(x_hbm.at[src_a], rows_v0, sem0).wait()
    plsc.subcore_barrier()

    pltpu.sync_copy(agg_sh.at[pl.ds(zbase, ROWS_PER_TILE)],
                    agg_out.at[cid, pl.ds(zbase, ROWS_PER_TILE)])


_sc_agg = pl.kernel(
    _sc_agg_body,
    out_type=[jax.ShapeDtypeStruct((NC, NP, D), jnp.float32)],
    mesh=_mesh,
    scratch_types=[
        pltpu.VMEM((KS, CHUNK), jnp.int32),
        pltpu.VMEM((CHUNK,), jnp.int32),
        pltpu.VMEM((CHUNK,), jnp.int32),
        pltpu.VMEM((CHUNK,), jnp.int32),
        pltpu.VMEM((CHUNK,), jnp.int32),
        pltpu.VMEM((CHUNK, D), jnp.float32),
        pltpu.VMEM((CHUNK, D), jnp.float32),
        pltpu.VMEM_SHARED((NP, D), jnp.float32),
        pltpu.SemaphoreType.DMA,
        pltpu.SemaphoreType.DMA,
    ])


def _sc_cnt_body(pk_hbm, zcnt_hbm, ones_hbm, cnt_out, pk_v, dst_a, ones_v,
                 cnt_sh, sem):
    cid = lax.axis_index("c")
    sid = lax.axis_index("s")
    wid = sid * NC + cid

    zbase = sid * ROWS_PER_TILE
    pltpu.sync_copy(zcnt_hbm.at[pl.ds(zbase, ROWS_PER_TILE)],
                    cnt_sh.at[pl.ds(zbase, ROWS_PER_TILE)])
    pltpu.sync_copy(ones_hbm, ones_v)
    pltpu.sync_copy(pk_hbm.at[wid], pk_v)
    plsc.subcore_barrier()

    def body(j, carry):
        _unpack_dst(pk_v, j, dst_a)
        pltpu.sync_copy(ones_v, cnt_sh.at[dst_a], add=True)
        return carry

    lax.fori_loop(0, KD, body, 0)
    plsc.subcore_barrier()

    pltpu.sync_copy(cnt_sh.at[pl.ds(zbase, ROWS_PER_TILE)],
                    cnt_out.at[cid, pl.ds(zbase, ROWS_PER_TILE)])


_sc_cnt = pl.kernel(
    _sc_cnt_body,
    out_type=[jax.ShapeDtypeStruct((NC, NP, CW), jnp.float32)],
    mesh=_mesh,
    scratch_types=[
        pltpu.VMEM((KS, CHUNK), jnp.int32),
        pltpu.VMEM((CHUNK,), jnp.int32),
        pltpu.VMEM((CHUNK, CW), jnp.float32),
        pltpu.VMEM_SHARED((NP, CW), jnp.float32),
        pltpu.SemaphoreType.DMA,
    ])

R = 1000


def _tc1_body(aggp_ref, cntp_ref, x_ref, wl_ref, bl_ref, wr_ref, wfc_ref,
              bfc_ref, h_ref, inp_ref):
    c = cntp_ref[0, :, :1] + cntp_ref[1, :, :1]
    agg = (aggp_ref[0] + aggp_ref[1]) / jnp.maximum(c, 1.0)
    x = x_ref[...]
    h = (jnp.dot(agg, wl_ref[...], preferred_element_type=jnp.float32)
         + bl_ref[...]
         + jnp.dot(x, wr_ref[...], preferred_element_type=jnp.float32))
    h_ref[...] = jnp.maximum(h, 0.0)
    inp_ref[...] = (jnp.dot(x, wfc_ref[...],
                            preferred_element_type=jnp.float32) + bfc_ref[...])


def _tc2_body(aggp_ref, cntp_ref, h_ref, wl_ref, bl_ref, wr_ref, inp_ref,
              out_ref):
    c = cntp_ref[0, :, :1] + cntp_ref[1, :, :1]
    agg = (aggp_ref[0] + aggp_ref[1]) / jnp.maximum(c, 1.0)
    o = (jnp.dot(agg, wl_ref[...], preferred_element_type=jnp.float32)
         + bl_ref[...]
         + jnp.dot(h_ref[...], wr_ref[...], preferred_element_type=jnp.float32)
         + inp_ref[...])
    out_ref[...] = jnp.maximum(o, 0.0)


_agg_spec = pl.BlockSpec((NC, R, D), lambda i: (0, i, 0))
_cnt_spec = pl.BlockSpec((NC, R, CW), lambda i: (0, i, 0))
_row_spec = pl.BlockSpec((R, D), lambda i: (i, 0))
_w_spec = pl.BlockSpec((D, D), lambda i: (0, 0))
_b_spec = pl.BlockSpec((1, D), lambda i: (0, 0))

_tc1 = pl.pallas_call(
    _tc1_body,
    grid=(N // R,),
    in_specs=[_agg_spec, _cnt_spec, _row_spec, _w_spec, _b_spec, _w_spec,
              _w_spec, _b_spec],
    out_specs=[_row_spec, _row_spec],
    out_shape=[jax.ShapeDtypeStruct((N, D), jnp.float32),
               jax.ShapeDtypeStruct((N, D), jnp.float32)],
)

_tc2 = pl.pallas_call(
    _tc2_body,
    grid=(N // R,),
    in_specs=[_agg_spec, _cnt_spec, _row_spec, _w_spec, _b_spec, _w_spec,
              _row_spec],
    out_specs=_row_spec,
    out_shape=jax.ShapeDtypeStruct((N, D), jnp.float32),
)


def _pack_edges(edge_index):
    src = edge_index[0]
    dst = edge_index[1]
    pad = E_PAD - E
    pk = (jnp.concatenate([src, jnp.zeros((pad,), jnp.int32)]) << PBITS) | \
        jnp.concatenate([dst, jnp.full((pad,), N, jnp.int32)])
    pk = pk.reshape(NW, K, CHUNK)
    return jnp.concatenate(
        [pk, jnp.full((NW, KS - K, CHUNK), N, jnp.int32)], axis=1)


def kernel(x, edge_index, Wl1, bl1, Wr1, Wl2, bl2, Wr2, Wfc, bfc):
    pk = _pack_edges(edge_index)
    zagg = jnp.zeros((NP, D), jnp.float32)
    zcnt = jnp.zeros((NP, CW), jnp.float32)
    ones = jnp.ones((CHUNK, CW), jnp.float32)

    (cntp,) = _sc_cnt(pk, zcnt, ones)
    (aggp1,) = _sc_agg(x, pk, zagg)
    h, inp = _tc1(aggp1, cntp, x, Wl1, bl1.reshape(1, D), Wr1, Wfc,
                  bfc.reshape(1, D))
    (aggp2,) = _sc_agg(h, pk, zagg)
    return _tc2(aggp2, cntp, h, Wl2, bl2.reshape(1, D), Wr2, inp)


# speedup vs baseline: 1.7448x; 1.7448x over previous
import functools

import jax
import jax.numpy as jnp
from jax import lax
from jax.experimental import pallas as pl
from jax.experimental.pallas import tpu as pltpu
from jax.experimental.pallas import tpu_sc as plsc

N = 10000
E = 320000
D = 128

NC = 2
NS = 16
NW = NC * NS

CHUNK = 128
K = 79
EPT = K * CHUNK
E_PAD = NW * EPT
NP = 10112
ROWS_PER_TILE = NP // NS
CW = 128

_mesh = plsc.VectorSubcoreMesh(
    core_axis_name="c", subcore_axis_name="s", num_cores=NC, num_subcores=NS)


def _sc_agg_body(x_hbm, src_hbm, dst_hbm, zagg_hbm, agg_out, src_v, dst_v,
                 rows_v, agg_sh, sems):
    cid = lax.axis_index("c")
    sid = lax.axis_index("s")
    wid = sid * NC + cid

    zbase = sid * ROWS_PER_TILE
    pltpu.sync_copy(zagg_hbm.at[pl.ds(zbase, ROWS_PER_TILE)],
                    agg_sh.at[pl.ds(zbase, ROWS_PER_TILE)])

    pltpu.sync_copy(src_hbm.at[wid], src_v)
    pltpu.sync_copy(dst_hbm.at[wid], dst_v)
    plsc.subcore_barrier()

    def body(j, carry):
        pltpu.async_copy(x_hbm.at[src_v.at[j]], rows_v, sems).wait()
        pltpu.sync_copy(rows_v, agg_sh.at[dst_v.at[j]], add=True)
        return carry

    lax.fori_loop(0, K, body, 0)
    plsc.subcore_barrier()

    pltpu.sync_copy(agg_sh.at[pl.ds(zbase, ROWS_PER_TILE)],
                    agg_out.at[cid, pl.ds(zbase, ROWS_PER_TILE)])


_sc_agg = pl.kernel(
    _sc_agg_body,
    out_type=[jax.ShapeDtypeStruct((NC, NP, D), jnp.float32)],
    mesh=_mesh,
    scratch_types=[
        pltpu.VMEM((K, CHUNK), jnp.int32),
        pltpu.VMEM((K, CHUNK), jnp.int32),
        pltpu.VMEM((CHUNK, D), jnp.float32),
        pltpu.VMEM_SHARED((NP, D), jnp.float32),
        pltpu.SemaphoreType.DMA,
    ])


def _sc_cnt_body(dst_hbm, zcnt_hbm, ones_hbm, cnt_out, dst_v, ones_v,
                 cnt_sh, sem):
    cid = lax.axis_index("c")
    sid = lax.axis_index("s")
    wid = sid * NC + cid

    zbase = sid * ROWS_PER_TILE
    pltpu.sync_copy(zcnt_hbm.at[pl.ds(zbase, ROWS_PER_TILE)],
                    cnt_sh.at[pl.ds(zbase, ROWS_PER_TILE)])
    pltpu.sync_copy(ones_hbm, ones_v)
    pltpu.sync_copy(dst_hbm.at[wid], dst_v)
    plsc.subcore_barrier()

    def body(j, carry):
        pltpu.sync_copy(ones_v, cnt_sh.at[dst_v.at[j]], add=True)
        return carry

    lax.fori_loop(0, K, body, 0)
    plsc.subcore_barrier()

    pltpu.sync_copy(cnt_sh.at[pl.ds(zbase, ROWS_PER_TILE)],
                    cnt_out.at[cid, pl.ds(zbase, ROWS_PER_TILE)])


_sc_cnt = pl.kernel(
    _sc_cnt_body,
    out_type=[jax.ShapeDtypeStruct((NC, NP, CW), jnp.float32)],
    mesh=_mesh,
    scratch_types=[
        pltpu.VMEM((K, CHUNK), jnp.int32),
        pltpu.VMEM((CHUNK, CW), jnp.float32),
        pltpu.VMEM_SHARED((NP, CW), jnp.float32),
        pltpu.SemaphoreType.DMA,
    ])

R = 1000


def _tc1_body(aggp_ref, cntp_ref, x_ref, wl_ref, bl_ref, wr_ref, wfc_ref,
              bfc_ref, h_ref, inp_ref):
    c = cntp_ref[0, :, :1] + cntp_ref[1, :, :1]
    agg = (aggp_ref[0] + aggp_ref[1]) / jnp.maximum(c, 1.0)
    x = x_ref[...]
    h = (jnp.dot(agg, wl_ref[...], preferred_element_type=jnp.float32)
         + bl_ref[...]
         + jnp.dot(x, wr_ref[...], preferred_element_type=jnp.float32))
    h_ref[...] = jnp.maximum(h, 0.0)
    inp_ref[...] = (jnp.dot(x, wfc_ref[...],
                            preferred_element_type=jnp.float32) + bfc_ref[...])


def _tc2_body(aggp_ref, cntp_ref, h_ref, wl_ref, bl_ref, wr_ref, inp_ref,
              out_ref):
    c = cntp_ref[0, :, :1] + cntp_ref[1, :, :1]
    agg = (aggp_ref[0] + aggp_ref[1]) / jnp.maximum(c, 1.0)
    o = (jnp.dot(agg, wl_ref[...], preferred_element_type=jnp.float32)
         + bl_ref[...]
         + jnp.dot(h_ref[...], wr_ref[...], preferred_element_type=jnp.float32)
         + inp_ref[...])
    out_ref[...] = jnp.maximum(o, 0.0)


_agg_spec = pl.BlockSpec((NC, R, D), lambda i: (0, i, 0))
_cnt_spec = pl.BlockSpec((NC, R, CW), lambda i: (0, i, 0))
_row_spec = pl.BlockSpec((R, D), lambda i: (i, 0))
_w_spec = pl.BlockSpec((D, D), lambda i: (0, 0))
_b_spec = pl.BlockSpec((1, D), lambda i: (0, 0))

_tc1 = pl.pallas_call(
    _tc1_body,
    grid=(N // R,),
    in_specs=[_agg_spec, _cnt_spec, _row_spec, _w_spec, _b_spec, _w_spec,
              _w_spec, _b_spec],
    out_specs=[_row_spec, _row_spec],
    out_shape=[jax.ShapeDtypeStruct((N, D), jnp.float32),
               jax.ShapeDtypeStruct((N, D), jnp.float32)],
)

_tc2 = pl.pallas_call(
    _tc2_body,
    grid=(N // R,),
    in_specs=[_agg_spec, _cnt_spec, _row_spec, _w_spec, _b_spec, _w_spec,
              _row_spec],
    out_specs=_row_spec,
    out_shape=jax.ShapeDtypeStruct((N, D), jnp.float32),
)


def _prep_edges(edge_index):
    src = edge_index[0]
    dst = edge_index[1]
    pad = E_PAD - E
    src_p = jnp.concatenate([src, jnp.zeros((pad,), jnp.int32)]
                            ).reshape(NW, K, CHUNK)
    dst_p = jnp.concatenate([dst, jnp.full((pad,), N, jnp.int32)]
                            ).reshape(NW, K, CHUNK)
    return src_p, dst_p


def kernel(x, edge_index, Wl1, bl1, Wr1, Wl2, bl2, Wr2, Wfc, bfc):
    src_p, dst_p = _prep_edges(edge_index)
    zagg = jnp.zeros((NP, D), jnp.float32)
    zcnt = jnp.zeros((NP, CW), jnp.float32)
    ones = jnp.ones((CHUNK, CW), jnp.float32)

    (cntp,) = _sc_cnt(dst_p, zcnt, ones)
    (aggp1,) = _sc_agg(x, src_p, dst_p, zagg)
    h, inp = _tc1(aggp1, cntp, x, Wl1, bl1.reshape(1, D), Wr1, Wfc,
                  bfc.reshape(1, D))
    (aggp2,) = _sc_agg(h, src_p, dst_p, zagg)
    return _tc2(aggp2, cntp, h, Wl2, bl2.reshape(1, D), Wr2, inp)
